# baseline (speedup 1.0000x reference)
import functools

import jax
import jax.numpy as jnp
from jax import lax
from jax.experimental import pallas as pl
from jax.experimental.pallas import tpu as pltpu
from jax.experimental.pallas import tpu_sc as plsc

BLK = 256
N_TOK = 4096
CAP = N_TOK + BLK
NBLK = CAP // BLK
NWAIT = 64
YROWS = 3 * CAP + NWAIT

_MESH = plsc.VectorSubcoreMesh(core_axis_name="c", subcore_axis_name="s")
_NC = 2
_NS = 16
_NW = _NC * _NS


def _dispatch_body(agv_i, mach_i, opf_i, opt_i, d_p, d_t, d_m,
                   agv_t, op_t, mach_t, wait2d,
                   xp, xt, xm, y_all,
                   idx_v, dp_v, dt_v, dm_v,
                   rows0_v, rows1_v, wbuf, gsem, ssem):
    wid = lax.axis_index("s") * _NC + lax.axis_index("c")
    per_w = N_TOK // _NW
    base = wid * per_w

    pltpu.sync_copy(wait2d, wbuf)
    pltpu.sync_copy(wbuf, y_all.at[pl.ds(3 * CAP + 2 * wid, 1)])
    pltpu.sync_copy(wbuf, y_all.at[pl.ds(3 * CAP + 2 * wid + 1, 1)])

    for dest_arr, dv in ((d_p, dp_v), (d_t, dt_v), (d_m, dm_v)):
        pltpu.sync_copy(dest_arr.at[pl.ds(base, per_w)], dv)

    plan = [
        (agv_i, agv_t, [(dp_v, xp, 0), (dt_v, xt, 0), (dm_v, xm, 0)]),
        (mach_i, mach_t, [(dp_v, xp, 768), (dt_v, xt, 256), (dm_v, xm, 256)]),
        (opf_i, op_t, [(dp_v, xp, 256)]),
        (opt_i, op_t, [(dp_v, xp, 512)]),
    ]
    bufs = [rows0_v, rows1_v]
    pending = []
    for e, (idx_arr, table, outs) in enumerate(plan):
        if e >= 2:
            for c in pending[e - 2]:
                c.wait()
        buf = bufs[e % 2]
        pltpu.sync_copy(idx_arr.at[pl.ds(base, per_w)], idx_v)
        pltpu.async_copy(table.at[idx_v], buf, gsem).wait()
        started = []
        for dv, out_ref, col in outs:
            started.append(pltpu.async_copy(
                buf, out_ref.at[dv, pl.ds(col, 256)], ssem))
        pending.append(started)
    for started in pending[2:]:
        for c in started:
            c.wait()


def _dispatch(agv_i, mach_i, opf_i, opt_i, dests, agv_t, op_t, mach_t,
              wait2d):
    per_w = N_TOK // _NW
    run = pl.kernel(
        _dispatch_body, mesh=_MESH,
        out_type=(
            jax.ShapeDtypeStruct((CAP, 1024), jnp.float32),
            jax.ShapeDtypeStruct((CAP, 512), jnp.float32),
            jax.ShapeDtypeStruct((CAP, 512), jnp.float32),
            jax.ShapeDtypeStruct((YROWS, 1024), jnp.float32),
        ),
        scratch_types=[
            pltpu.VMEM((per_w,), jnp.int32),
            pltpu.VMEM((per_w,), jnp.int32),
            pltpu.VMEM((per_w,), jnp.int32),
            pltpu.VMEM((per_w,), jnp.int32),
            pltpu.VMEM((per_w, 256), jnp.float32),
            pltpu.VMEM((per_w, 256), jnp.float32),
            pltpu.VMEM((1, 1024), jnp.float32),
            pltpu.SemaphoreType.DMA,
            pltpu.SemaphoreType.DMA,
        ],
    )
    return run(agv_i, mach_i, opf_i, opt_i, *dests, agv_t, op_t, mach_t,
               wait2d)


def _combine_body(y_all, src, out, sidx_v, ybuf, sem):
    wid = lax.axis_index("s") * _NC + lax.axis_index("c")
    per_w = N_TOK // _NW
    half = per_w // 2
    for h in range(2):
        b2 = wid * per_w + h * half
        pltpu.sync_copy(src.at[pl.ds(b2, half)], sidx_v)
        pltpu.async_copy(y_all.at[sidx_v], ybuf, sem).wait()
        pltpu.sync_copy(ybuf, out.at[pl.ds(b2, half)])


def _combine(y_all, src):
    half = N_TOK // _NW // 2
    run = pl.kernel(
        _combine_body, mesh=_MESH,
        out_type=jax.ShapeDtypeStruct((N_TOK, 1024), jnp.float32),
        scratch_types=[
            pltpu.VMEM((half,), jnp.int32),
            pltpu.VMEM((half, 1024), jnp.float32),
            pltpu.SemaphoreType.DMA,
        ],
    )
    return run(y_all, src)


def _route_body(t_ref, d_p, d_t, d_m, src_ref, nts_ref):
    tt = t_ref[...]
    ii = lax.broadcasted_iota(jnp.int32, (128, 128), 0)
    jj = lax.broadcasted_iota(jnp.int32, (128, 128), 1)
    E = (ii < jj).astype(jnp.float32)
    i2 = lax.broadcasted_iota(jnp.int32, (32, 32), 0)
    j2 = lax.broadcasted_iota(jnp.int32, (32, 32), 1)
    P = (i2 < j2).astype(jnp.float32)

    def rank_of(mask):
        m = mask.astype(jnp.float32)
        within = jnp.dot(m, E, preferred_element_type=jnp.float32)
        rowtot = jnp.sum(m, axis=1).reshape(1, 32)
        rowpre = jnp.dot(rowtot, P,
                         preferred_element_type=jnp.float32).reshape(32, 1)
        return (within + rowpre).astype(jnp.int32), jnp.sum(m).astype(jnp.int32)

    is0, is1, is2, is3 = (tt == 0), (tt == 1), (tt == 2), (tt == 3)
    r0, _ = rank_of(is0)
    r1, c1 = rank_of(is1)
    r2, c2 = rank_of(is2)
    r3, c3 = rank_of(is3)

    ar = (lax.broadcasted_iota(jnp.int32, (32, 128), 0) * 128
          + lax.broadcasted_iota(jnp.int32, (32, 128), 1))
    dump = N_TOK + ar % (CAP - N_TOK)
    d_p[...] = jnp.where(is1, r1, dump)
    d_t[...] = jnp.where(is2, r2, dump)
    d_m[...] = jnp.where(is3, r3, dump)
    src_ref[...] = jnp.where(is1, r1,
                             jnp.where(is2, CAP + r2,
                                       jnp.where(is3, 2 * CAP + r3,
                                                 3 * CAP + r0 % NWAIT)))
    lane = lax.broadcasted_iota(jnp.int32, (1, 128), 1)
    nt1 = (c1 + BLK - 1) // BLK
    nt2 = (c2 + BLK - 1) // BLK
    nt3 = (c3 + BLK - 1) // BLK
    nts_ref[...] = jnp.where(lane == 0, nt1,
                             jnp.where(lane == 1, nt2,
                                       jnp.where(lane == 2, nt3, 0)))


def _route(t2d):
    shp = jax.ShapeDtypeStruct((32, 128), jnp.int32)
    outs = pl.pallas_call(
        _route_body,
        out_shape=[shp] * 4 + [jax.ShapeDtypeStruct((1, 128), jnp.int32)],
    )(t2d)
    return [o.reshape(N_TOK) for o in outs[:4]] + [outs[4]]


def _dot(x, wref):
    return jnp.dot(x.astype(jnp.bfloat16), wref[...],
                   preferred_element_type=jnp.float32)


def _expert_chain(x, wrefs, b):
    proj = len(wrefs) == 9
    h = jnp.tanh(_dot(x, wrefs[0]) + b[0])
    y = _dot(h, wrefs[1]) + b[1]
    res = _dot(x, wrefs[2]) + b[2] if proj else x
    x = jnp.tanh(res + y)
    off = 3 if proj else 2
    for k in range(2):
        wa, wb = wrefs[off + 2 * k], wrefs[off + 2 * k + 1]
        h = jnp.tanh(_dot(x, wa) + b[off + 2 * k])
        x = jnp.tanh(x + _dot(h, wb) + b[off + 2 * k + 1])
    wa, wb = wrefs[off + 4], wrefs[off + 5]
    h = jnp.tanh(_dot(x, wa) + b[off + 4])
    return x + _dot(h, wb) + b[off + 5]


def _expert_body(nt_ref, x_ref, *rest):
    *wrefs, b_ref, yin_ref, o_ref = rest
    del yin_ref
    i = pl.program_id(0)

    @pl.when(i < nt_ref[0])
    def _():
        o_ref[...] = _expert_chain(x_ref[...], list(wrefs), b_ref[...])


def _run_expert(x, ws, bs, nt, y_all, blk_off):
    cap, din = x.shape
    nmat = len(ws)
    bstack = jnp.stack(bs)

    def clamp(i, s):
        return jnp.clip(i, 0, jnp.maximum(s[0] - 1, 0))

    in_specs = [pl.BlockSpec((BLK, din), lambda i, s: (clamp(i, s), 0))]
    for w in ws:
        in_specs.append(pl.BlockSpec(w.shape, lambda i, s: (0,) * w.ndim))
    in_specs.append(pl.BlockSpec((nmat, 1024), lambda i, s: (0, 0)))
    in_specs.append(pl.BlockSpec((8, 1024), lambda i, s: (0, 0)))
    return pl.pallas_call(
        _expert_body,
        grid_spec=pltpu.PrefetchScalarGridSpec(
            num_scalar_prefetch=1,
            grid=(cap // BLK,),
            in_specs=in_specs,
            out_specs=pl.BlockSpec((BLK, 1024),
                                   lambda i, s: (clamp(i, s) + blk_off, 0)),
        ),
        out_shape=jax.ShapeDtypeStruct((YROWS, 1024), jnp.float32),
        input_output_aliases={2 + nmat + 1: 0},
    )(nt, x, *ws, bstack, y_all)


def _enc_weights(p, proj):
    w = [p["first"]["W1"], p["first"]["W2"]]
    b = [p["first"]["b1"], p["first"]["b2"]]
    if proj:
        w.append(p["first"]["Wp"])
        b.append(p["first"]["bp"])
    for sp in p["stack"]:
        w += [sp["W1"], sp["W2"]]
        b += [sp["b1"], sp["b2"]]
    w += [p["last"]["W1"], p["last"]["W2"]]
    b += [p["last"]["b1"], p["last"]["b2"]]
    return [wi.astype(jnp.bfloat16) for wi in w], b


def kernel(action_type, AGV_idx, op_from_idx, op_to_idx, machine_idx,
           AGV_emb, operation_emb, machine_emb, wait_emb,
           pick_params, transport_params, move_params):
    *dests_src, nts = _route(action_type.astype(jnp.int32).reshape(32, 128))
    dests, srcrow = dests_src[:3], dests_src[3]

    Xp, Xt, Xm, y0 = _dispatch(
        AGV_idx.astype(jnp.int32), machine_idx.astype(jnp.int32),
        op_from_idx.astype(jnp.int32), op_to_idx.astype(jnp.int32),
        dests, AGV_emb, operation_emb, machine_emb,
        wait_emb.reshape(1, 1024))

    wp, bp = _enc_weights(pick_params, proj=False)
    wt, bt = _enc_weights(transport_params, proj=True)
    wm, bm = _enc_weights(move_params, proj=True)

    y1 = _run_expert(Xp, wp, bp, nts[0, 0:1], y0, 0)
    y2 = _run_expert(Xt, wt, bt, nts[0, 1:2], y1, NBLK)
    y3 = _run_expert(Xm, wm, bm, nts[0, 2:3], y2, 2 * NBLK)

    return _combine(y3, srcrow)


# speedup vs baseline: 1.0228x; 1.0228x over previous
import functools

import jax
import jax.numpy as jnp
from jax import lax
from jax.experimental import pallas as pl
from jax.experimental.pallas import tpu as pltpu
from jax.experimental.pallas import tpu_sc as plsc

BLK = 256
N_TOK = 4096
CAP = N_TOK + BLK
NBLK = CAP // BLK
NWAIT = 64
YROWS = 3 * CAP + NWAIT

_MESH = plsc.VectorSubcoreMesh(core_axis_name="c", subcore_axis_name="s")
_NC = 2
_NS = 16
_NW = _NC * _NS


def _dispatch_body(agv_i, mach_i, opf_i, opt_i, d_a, d_p, d_t,
                   agv_t, op_t, mach_t, wait2d,
                   x_all, y_all,
                   idx_v, da_v, dp_v, dt_v,
                   rows0_v, rows1_v, wbuf, gsem, ssem):
    wid = lax.axis_index("s") * _NC + lax.axis_index("c")
    per_w = N_TOK // _NW
    base = wid * per_w

    pltpu.sync_copy(wait2d, wbuf)
    pltpu.sync_copy(wbuf, y_all.at[pl.ds(3 * CAP + 2 * wid, 1)])
    pltpu.sync_copy(wbuf, y_all.at[pl.ds(3 * CAP + 2 * wid + 1, 1)])

    for dest_arr, dv in ((d_a, da_v), (d_p, dp_v), (d_t, dt_v)):
        pltpu.sync_copy(dest_arr.at[pl.ds(base, per_w)], dv)

    plan = [
        (agv_i, agv_t, [(da_v, 0)]),
        (mach_i, mach_t, [(dp_v, 768), (dt_v, 256)]),
        (opf_i, op_t, [(dp_v, 256)]),
        (opt_i, op_t, [(dp_v, 512)]),
    ]
    bufs = [rows0_v, rows1_v]
    pending = []
    for e, (idx_arr, table, outs) in enumerate(plan):
        if e >= 2:
            for c in pending[e - 2]:
                c.wait()
        buf = bufs[e % 2]
        pltpu.sync_copy(idx_arr.at[pl.ds(base, per_w)], idx_v)
        pltpu.async_copy(table.at[idx_v], buf, gsem).wait()
        started = []
        for dv, col in outs:
            started.append(pltpu.async_copy(
                buf, x_all.at[dv, pl.ds(col, 256)], ssem))
        pending.append(started)
    for started in pending[2:]:
        for c in started:
            c.wait()


def _dispatch(agv_i, mach_i, opf_i, opt_i, dests, agv_t, op_t, mach_t,
              wait2d):
    per_w = N_TOK // _NW
    run = pl.kernel(
        _dispatch_body, mesh=_MESH,
        out_type=(
            jax.ShapeDtypeStruct((3 * CAP, 1024), jnp.float32),
            jax.ShapeDtypeStruct((YROWS, 1024), jnp.float32),
        ),
        scratch_types=[
            pltpu.VMEM((per_w,), jnp.int32),
            pltpu.VMEM((per_w,), jnp.int32),
            pltpu.VMEM((per_w,), jnp.int32),
            pltpu.VMEM((per_w,), jnp.int32),
            pltpu.VMEM((per_w, 256), jnp.float32),
            pltpu.VMEM((per_w, 256), jnp.float32),
            pltpu.VMEM((1, 1024), jnp.float32),
            pltpu.SemaphoreType.DMA,
            pltpu.SemaphoreType.DMA,
        ],
    )
    return run(agv_i, mach_i, opf_i, opt_i, *dests, agv_t, op_t, mach_t,
               wait2d)


def _combine_body(y_all, src, out, sidx_v, ybuf, sem):
    wid = lax.axis_index("s") * _NC + lax.axis_index("c")
    per_w = N_TOK // _NW
    half = per_w // 2
    for h in range(2):
        b2 = wid * per_w + h * half
        pltpu.sync_copy(src.at[pl.ds(b2, half)], sidx_v)
        pltpu.async_copy(y_all.at[sidx_v], ybuf, sem).wait()
        pltpu.sync_copy(ybuf, out.at[pl.ds(b2, half)])


def _combine(y_all, src):
    half = N_TOK // _NW // 2
    run = pl.kernel(
        _combine_body, mesh=_MESH,
        out_type=jax.ShapeDtypeStruct((N_TOK, 1024), jnp.float32),
        scratch_types=[
            pltpu.VMEM((half,), jnp.int32),
            pltpu.VMEM((half, 1024), jnp.float32),
            pltpu.SemaphoreType.DMA,
        ],
    )
    return run(y_all, src)


def _route_body(t_ref, d_a, d_p, d_t, src_ref, nts_ref):
    tt = t_ref[...]
    ii = lax.broadcasted_iota(jnp.int32, (128, 128), 0)
    jj = lax.broadcasted_iota(jnp.int32, (128, 128), 1)
    E = (ii < jj).astype(jnp.float32)
    i2 = lax.broadcasted_iota(jnp.int32, (32, 32), 0)
    j2 = lax.broadcasted_iota(jnp.int32, (32, 32), 1)
    P = (i2 < j2).astype(jnp.float32)

    def rank_of(mask):
        m = mask.astype(jnp.float32)
        within = jnp.dot(m, E, preferred_element_type=jnp.float32)
        rowtot = jnp.sum(m, axis=1).reshape(1, 32)
        rowpre = jnp.dot(rowtot, P,
                         preferred_element_type=jnp.float32).reshape(32, 1)
        return (within + rowpre).astype(jnp.int32), jnp.sum(m).astype(jnp.int32)

    is0, is1, is2, is3 = (tt == 0), (tt == 1), (tt == 2), (tt == 3)
    r0, _ = rank_of(is0)
    r1, c1 = rank_of(is1)
    r2, c2 = rank_of(is2)
    r3, c3 = rank_of(is3)

    ar = (lax.broadcasted_iota(jnp.int32, (32, 128), 0) * 128
          + lax.broadcasted_iota(jnp.int32, (32, 128), 1))
    dump = (ar % 3) * CAP + N_TOK + ar % (CAP - N_TOK)
    d_a[...] = jnp.where(is1, r1,
                         jnp.where(is2, CAP + r2,
                                   jnp.where(is3, 2 * CAP + r3, dump)))
    d_p[...] = jnp.where(is1, r1, dump)
    d_t[...] = jnp.where(is2, CAP + r2,
                         jnp.where(is3, 2 * CAP + r3, dump))
    src_ref[...] = jnp.where(is1, r1,
                             jnp.where(is2, CAP + r2,
                                       jnp.where(is3, 2 * CAP + r3,
                                                 3 * CAP + r0 % NWAIT)))
    lane = lax.broadcasted_iota(jnp.int32, (1, 128), 1)
    nt1 = (c1 + BLK - 1) // BLK
    nt2 = (c2 + BLK - 1) // BLK
    nt3 = (c3 + BLK - 1) // BLK
    nts_ref[...] = jnp.where(lane == 0, nt1,
                             jnp.where(lane == 1, nt2,
                                       jnp.where(lane == 2, nt3, 0)))


def _route(t2d):
    shp = jax.ShapeDtypeStruct((32, 128), jnp.int32)
    outs = pl.pallas_call(
        _route_body,
        out_shape=[shp] * 4 + [jax.ShapeDtypeStruct((1, 128), jnp.int32)],
    )(t2d)
    return [o.reshape(N_TOK) for o in outs[:4]] + [outs[4]]


def _dot(x, wref):
    return jnp.dot(x.astype(jnp.bfloat16), wref[...],
                   preferred_element_type=jnp.float32)


def _expert_chain(x, wrefs, b):
    proj = len(wrefs) == 9
    h = jnp.tanh(_dot(x, wrefs[0]) + b[0])
    y = _dot(h, wrefs[1]) + b[1]
    res = _dot(x, wrefs[2]) + b[2] if proj else x
    x = jnp.tanh(res + y)
    off = 3 if proj else 2
    for k in range(2):
        wa, wb = wrefs[off + 2 * k], wrefs[off + 2 * k + 1]
        h = jnp.tanh(_dot(x, wa) + b[off + 2 * k])
        x = jnp.tanh(x + _dot(h, wb) + b[off + 2 * k + 1])
    wa, wb = wrefs[off + 4], wrefs[off + 5]
    h = jnp.tanh(_dot(x, wa) + b[off + 4])
    return x + _dot(h, wb) + b[off + 5]


def _expert_body(nt_ref, x_ref, *rest):
    *wrefs, b_ref, yin_ref, o_ref = rest
    del yin_ref
    i = pl.program_id(0)

    @pl.when(i < nt_ref[0])
    def _():
        o_ref[...] = _expert_chain(x_ref[...], list(wrefs), b_ref[...])


def _run_expert(x_all, ws, bs, nt, y_all, blk_off):
    din = ws[0].shape[0]
    nmat = len(ws)
    bstack = jnp.stack(bs)

    def clamp(i, s):
        return jnp.clip(i, 0, jnp.maximum(s[0] - 1, 0))

    in_specs = [pl.BlockSpec((BLK, din),
                             lambda i, s: (clamp(i, s) + blk_off, 0))]
    for w in ws:
        in_specs.append(pl.BlockSpec(w.shape, lambda i, s: (0,) * w.ndim))
    in_specs.append(pl.BlockSpec((nmat, 1024), lambda i, s: (0, 0)))
    in_specs.append(pl.BlockSpec((8, 1024), lambda i, s: (0, 0)))
    return pl.pallas_call(
        _expert_body,
        grid_spec=pltpu.PrefetchScalarGridSpec(
            num_scalar_prefetch=1,
            grid=(NBLK,),
            in_specs=in_specs,
            out_specs=pl.BlockSpec((BLK, 1024),
                                   lambda i, s: (clamp(i, s) + blk_off, 0)),
        ),
        out_shape=jax.ShapeDtypeStruct((YROWS, 1024), jnp.float32),
        input_output_aliases={2 + nmat + 1: 0},
    )(nt, x_all, *ws, bstack, y_all)


def _enc_weights(p, proj):
    w = [p["first"]["W1"], p["first"]["W2"]]
    b = [p["first"]["b1"], p["first"]["b2"]]
    if proj:
        w.append(p["first"]["Wp"])
        b.append(p["first"]["bp"])
    for sp in p["stack"]:
        w += [sp["W1"], sp["W2"]]
        b += [sp["b1"], sp["b2"]]
    w += [p["last"]["W1"], p["last"]["W2"]]
    b += [p["last"]["b1"], p["last"]["b2"]]
    return [wi.astype(jnp.bfloat16) for wi in w], b


def kernel(action_type, AGV_idx, op_from_idx, op_to_idx, machine_idx,
           AGV_emb, operation_emb, machine_emb, wait_emb,
           pick_params, transport_params, move_params):
    *dests_src, nts = _route(action_type.astype(jnp.int32).reshape(32, 128))
    dests, srcrow = dests_src[:3], dests_src[3]

    X_all, y0 = _dispatch(
        AGV_idx.astype(jnp.int32), machine_idx.astype(jnp.int32),
        op_from_idx.astype(jnp.int32), op_to_idx.astype(jnp.int32),
        dests, AGV_emb, operation_emb, machine_emb,
        wait_emb.reshape(1, 1024))

    wp, bp = _enc_weights(pick_params, proj=False)
    wt, bt = _enc_weights(transport_params, proj=True)
    wm, bm = _enc_weights(move_params, proj=True)

    y1 = _run_expert(X_all, wp, bp, nts[0, 0:1], y0, 0)
    y2 = _run_expert(X_all, wt, bt, nts[0, 1:2], y1, NBLK)
    y3 = _run_expert(X_all, wm, bm, nts[0, 2:3], y2, 2 * NBLK)

    return _combine(y3, srcrow)


# speedup vs baseline: 1.3164x; 1.2871x over previous
import functools

import jax
import jax.numpy as jnp
from jax import lax
from jax.experimental import pallas as pl
from jax.experimental.pallas import tpu as pltpu
from jax.experimental.pallas import tpu_sc as plsc

BLK = 256
N_TOK = 4096
CAP = N_TOK + BLK
NBLK = CAP // BLK
NWAIT = 64
YROWS = 3 * CAP + NWAIT

_MESH = plsc.VectorSubcoreMesh(core_axis_name="c", subcore_axis_name="s")
_NC = 2
_NS = 16
_NW = _NC * _NS


def _dispatch_body(agv_i, mach_i, opf_i, opt_i, d_a, d_p, d_t,
                   agv_t, op_t, mach_t, wait2d,
                   x_all, y_all,
                   idx_v, da_v, dp_v, dt_v,
                   rows0_v, rows1_v, wbuf, gsem, ssem):
    wid = lax.axis_index("s") * _NC + lax.axis_index("c")
    per_w = N_TOK // _NW
    base = wid * per_w

    pltpu.sync_copy(wait2d, wbuf)
    pltpu.sync_copy(wbuf, y_all.at[pl.ds(3 * CAP + 2 * wid, 1)])
    pltpu.sync_copy(wbuf, y_all.at[pl.ds(3 * CAP + 2 * wid + 1, 1)])

    for dest_arr, dv in ((d_a, da_v), (d_p, dp_v), (d_t, dt_v)):
        pltpu.sync_copy(dest_arr.at[pl.ds(base, per_w)], dv)

    plan = [
        (agv_i, agv_t, [(da_v, 0)]),
        (mach_i, mach_t, [(dp_v, 768), (dt_v, 256)]),
        (opf_i, op_t, [(dp_v, 256)]),
        (opt_i, op_t, [(dp_v, 512)]),
    ]
    bufs = [rows0_v, rows1_v]
    pending = []
    for e, (idx_arr, table, outs) in enumerate(plan):
        if e >= 2:
            for c in pending[e - 2]:
                c.wait()
        buf = bufs[e % 2]
        pltpu.sync_copy(idx_arr.at[pl.ds(base, per_w)], idx_v)
        pltpu.async_copy(table.at[idx_v], buf, gsem).wait()
        started = []
        for dv, col in outs:
            started.append(pltpu.async_copy(
                buf, x_all.at[dv, pl.ds(col, 256)], ssem))
        pending.append(started)
    for started in pending[2:]:
        for c in started:
            c.wait()


def _dispatch(agv_i, mach_i, opf_i, opt_i, dests, agv_t, op_t, mach_t,
              wait2d):
    per_w = N_TOK // _NW
    run = pl.kernel(
        _dispatch_body, mesh=_MESH,
        out_type=(
            jax.ShapeDtypeStruct((3 * CAP, 1024), jnp.float32),
            jax.ShapeDtypeStruct((YROWS, 1024), jnp.float32),
        ),
        scratch_types=[
            pltpu.VMEM((per_w,), jnp.int32),
            pltpu.VMEM((per_w,), jnp.int32),
            pltpu.VMEM((per_w,), jnp.int32),
            pltpu.VMEM((per_w,), jnp.int32),
            pltpu.VMEM((per_w, 256), jnp.float32),
            pltpu.VMEM((per_w, 256), jnp.float32),
            pltpu.VMEM((1, 1024), jnp.float32),
            pltpu.SemaphoreType.DMA,
            pltpu.SemaphoreType.DMA,
        ],
    )
    return run(agv_i, mach_i, opf_i, opt_i, *dests, agv_t, op_t, mach_t,
               wait2d)


def _combine_body(y_all, src, out, sidx_v, ybuf, sem):
    wid = lax.axis_index("s") * _NC + lax.axis_index("c")
    per_w = N_TOK // _NW
    half = per_w // 2
    for h in range(2):
        b2 = wid * per_w + h * half
        pltpu.sync_copy(src.at[pl.ds(b2, half)], sidx_v)
        pltpu.async_copy(y_all.at[sidx_v], ybuf, sem).wait()
        pltpu.sync_copy(ybuf, out.at[pl.ds(b2, half)])


def _combine(y_all, src):
    half = N_TOK // _NW // 2
    run = pl.kernel(
        _combine_body, mesh=_MESH,
        out_type=jax.ShapeDtypeStruct((N_TOK, 1024), jnp.float32),
        scratch_types=[
            pltpu.VMEM((half,), jnp.int32),
            pltpu.VMEM((half, 1024), jnp.float32),
            pltpu.SemaphoreType.DMA,
        ],
    )
    return run(y_all, src)


def _route_body(t_ref, d_a, d_p, d_t, src_ref, nts_ref):
    tt = t_ref[...]
    ii = lax.broadcasted_iota(jnp.int32, (128, 128), 0)
    jj = lax.broadcasted_iota(jnp.int32, (128, 128), 1)
    E = (ii < jj).astype(jnp.float32)
    i2 = lax.broadcasted_iota(jnp.int32, (32, 32), 0)
    j2 = lax.broadcasted_iota(jnp.int32, (32, 32), 1)
    P = (i2 < j2).astype(jnp.float32)

    def rank_of(mask):
        m = mask.astype(jnp.float32)
        within = jnp.dot(m, E, preferred_element_type=jnp.float32)
        rowtot = jnp.sum(m, axis=1).reshape(1, 32)
        rowpre = jnp.dot(rowtot, P,
                         preferred_element_type=jnp.float32).reshape(32, 1)
        return (within + rowpre).astype(jnp.int32), jnp.sum(m).astype(jnp.int32)

    is0, is1, is2, is3 = (tt == 0), (tt == 1), (tt == 2), (tt == 3)
    r0, _ = rank_of(is0)
    r1, c1 = rank_of(is1)
    r2, c2 = rank_of(is2)
    r3, c3 = rank_of(is3)

    ar = (lax.broadcasted_iota(jnp.int32, (32, 128), 0) * 128
          + lax.broadcasted_iota(jnp.int32, (32, 128), 1))
    dump = (ar % 3) * CAP + N_TOK + ar % (CAP - N_TOK)
    d_a[...] = jnp.where(is1, r1,
                         jnp.where(is2, CAP + r2,
                                   jnp.where(is3, 2 * CAP + r3, dump)))
    d_p[...] = jnp.where(is1, r1, dump)
    d_t[...] = jnp.where(is2, CAP + r2,
                         jnp.where(is3, 2 * CAP + r3, dump))
    src_ref[...] = jnp.where(is1, r1,
                             jnp.where(is2, CAP + r2,
                                       jnp.where(is3, 2 * CAP + r3,
                                                 3 * CAP + r0 % NWAIT)))
    lane = lax.broadcasted_iota(jnp.int32, (1, 128), 1)
    nt1 = (c1 + BLK - 1) // BLK
    nt2 = (c2 + BLK - 1) // BLK
    nt3 = (c3 + BLK - 1) // BLK
    nts_ref[...] = jnp.where(lane == 0, nt1,
                             jnp.where(lane == 1, nt2,
                                       jnp.where(lane == 2, nt3, 0)))


def _route(t2d):
    shp = jax.ShapeDtypeStruct((32, 128), jnp.int32)
    outs = pl.pallas_call(
        _route_body,
        out_shape=[shp] * 4 + [jax.ShapeDtypeStruct((1, 128), jnp.int32)],
    )(t2d)
    return [o.reshape(N_TOK) for o in outs[:4]] + [outs[4]]


def _dot(x, wref):
    return jnp.dot(x, wref[...], precision=lax.Precision.DEFAULT,
                   preferred_element_type=jnp.float32)


def _expert_chain(x, wrefs, b):
    proj = len(wrefs) == 9
    h = jnp.tanh(_dot(x, wrefs[0]) + b[0])
    y = _dot(h, wrefs[1]) + b[1]
    res = _dot(x, wrefs[2]) + b[2] if proj else x
    x = jnp.tanh(res + y)
    off = 3 if proj else 2
    for k in range(2):
        wa, wb = wrefs[off + 2 * k], wrefs[off + 2 * k + 1]
        h = jnp.tanh(_dot(x, wa) + b[off + 2 * k])
        x = jnp.tanh(x + _dot(h, wb) + b[off + 2 * k + 1])
    wa, wb = wrefs[off + 4], wrefs[off + 5]
    h = jnp.tanh(_dot(x, wa) + b[off + 4])
    return x + _dot(h, wb) + b[off + 5]


def _expert_body(nt_ref, x_ref, *rest):
    *wrefs, b_ref, yin_ref, o_ref = rest
    del yin_ref
    i = pl.program_id(0)

    @pl.when(i < nt_ref[0])
    def _():
        o_ref[...] = _expert_chain(x_ref[...], list(wrefs), b_ref[...])


def _run_expert(x_all, ws, bs, nt, y_all, blk_off):
    din = ws[0].shape[0]
    nmat = len(ws)
    bstack = jnp.stack(bs)

    def clamp(i, s):
        return jnp.clip(i, 0, jnp.maximum(s[0] - 1, 0))

    in_specs = [pl.BlockSpec((BLK, din),
                             lambda i, s: (clamp(i, s) + blk_off, 0))]
    for w in ws:
        in_specs.append(pl.BlockSpec(w.shape, lambda i, s: (0,) * w.ndim))
    in_specs.append(pl.BlockSpec((nmat, 1024), lambda i, s: (0, 0)))
    in_specs.append(pl.BlockSpec((8, 1024), lambda i, s: (0, 0)))
    return pl.pallas_call(
        _expert_body,
        grid_spec=pltpu.PrefetchScalarGridSpec(
            num_scalar_prefetch=1,
            grid=(NBLK,),
            in_specs=in_specs,
            out_specs=pl.BlockSpec((BLK, 1024),
                                   lambda i, s: (clamp(i, s) + blk_off, 0)),
        ),
        out_shape=jax.ShapeDtypeStruct((YROWS, 1024), jnp.float32),
        input_output_aliases={2 + nmat + 1: 0},
    )(nt, x_all, *ws, bstack, y_all)


def _enc_weights(p, proj):
    w = [p["first"]["W1"], p["first"]["W2"]]
    b = [p["first"]["b1"], p["first"]["b2"]]
    if proj:
        w.append(p["first"]["Wp"])
        b.append(p["first"]["bp"])
    for sp in p["stack"]:
        w += [sp["W1"], sp["W2"]]
        b += [sp["b1"], sp["b2"]]
    w += [p["last"]["W1"], p["last"]["W2"]]
    b += [p["last"]["b1"], p["last"]["b2"]]
    return w, b


def kernel(action_type, AGV_idx, op_from_idx, op_to_idx, machine_idx,
           AGV_emb, operation_emb, machine_emb, wait_emb,
           pick_params, transport_params, move_params):
    *dests_src, nts = _route(action_type.astype(jnp.int32).reshape(32, 128))
    dests, srcrow = dests_src[:3], dests_src[3]

    X_all, y0 = _dispatch(
        AGV_idx.astype(jnp.int32), machine_idx.astype(jnp.int32),
        op_from_idx.astype(jnp.int32), op_to_idx.astype(jnp.int32),
        dests, AGV_emb, operation_emb, machine_emb,
        wait_emb.reshape(1, 1024))

    wp, bp = _enc_weights(pick_params, proj=False)
    wt, bt = _enc_weights(transport_params, proj=True)
    wm, bm = _enc_weights(move_params, proj=True)

    y1 = _run_expert(X_all, wp, bp, nts[0, 0:1], y0, 0)
    y2 = _run_expert(X_all, wt, bt, nts[0, 1:2], y1, NBLK)
    y3 = _run_expert(X_all, wm, bm, nts[0, 2:3], y2, 2 * NBLK)

    return _combine(y3, srcrow)
